# Initial kernel scaffold; baseline (speedup 1.0000x reference)
import jax
import jax.numpy as jnp
from jax.experimental import pallas as pl


def kernel(A, W1, b1, W2, b2, Wl, bl):
    raise NotImplementedError("write your pallas kernel here")


# speedup vs baseline: 1.8851x; 1.8851x over previous
import jax
import jax.numpy as jnp
from jax.experimental import pallas as pl


def _gcn_kernel(a_ref, w1_ref, b1_ref, w2_ref, b2_ref, wl_ref, bl_ref, out_ref):
    a = a_ref[0]
    deg = jnp.sum(a, axis=0)
    d = jnp.where(deg > 0, 1.0 / jnp.sqrt(deg), 0.0)
    s = jnp.sum(d[:, None] * a, axis=0)
    c = d * s
    h1 = jnp.maximum(c[:, None] * w1_ref[0][None, :] + b1_ref[0][None, :], 0.0)
    m = jnp.dot(h1, w2_ref[...], preferred_element_type=jnp.float32)
    p = d[:, None] * m
    t = jax.lax.dot_general(a, p, (((0,), (0,)), ((), ())),
                            preferred_element_type=jnp.float32)
    h2 = jnp.maximum(d[:, None] * t + b2_ref[0][None, :], 0.0)
    g = jnp.mean(h2, axis=0)
    logit = jnp.sum(g * wl_ref[...][:, 0]) + bl_ref[0, 0]
    out_ref[...] = jnp.reshape(logit, (1, 1, 1))


def kernel(A, W1, b1, W2, b2, Wl, bl):
    B, N, _ = A.shape
    H = W1.shape[1]
    out = pl.pallas_call(
        _gcn_kernel,
        grid=(B,),
        in_specs=[
            pl.BlockSpec((1, N, N), lambda b: (b, 0, 0)),
            pl.BlockSpec((1, H), lambda b: (0, 0)),
            pl.BlockSpec((1, H), lambda b: (0, 0)),
            pl.BlockSpec((H, H), lambda b: (0, 0)),
            pl.BlockSpec((1, H), lambda b: (0, 0)),
            pl.BlockSpec((H, 1), lambda b: (0, 0)),
            pl.BlockSpec((1, 1), lambda b: (0, 0)),
        ],
        out_specs=pl.BlockSpec((1, 1, 1), lambda b: (b, 0, 0)),
        out_shape=jax.ShapeDtypeStruct((B, 1, 1), jnp.float32),
    )(A, W1, b1.reshape(1, H), W2, b2.reshape(1, H), Wl, bl.reshape(1, 1))
    return out
